# baseline (device time: 15881 ns/iter reference)
import jax
import jax.numpy as jnp
from jax import lax
from jax.experimental import pallas as pl
from jax.experimental.pallas import tpu as pltpu

N_DEV = 8
_MASKS = (1, 3, 4)
_CHUNKS = ((0, 96), (96, 80), (176, 80))
_SCHED = ((1, 3, 4), (3, 4, 1), (4, 1, 3))
_N_STREAMS = len(_SCHED)
_N_ROUNDS = len(_MASKS)
_MAX_ROWS = max(r for _, r in _CHUNKS)


def kernel(x, Wg, Wu, Wd):
    m, k = x.shape
    d = Wd.shape[1]

    def body(x_ref, wg_ref, wu_ref, wd_ref, out_ref, stage_ref, comm_ref,
             send_sems, recv_sems):
        my = lax.axis_index("i")

        barrier_sem = pltpu.get_barrier_semaphore()
        for mask in _MASKS:
            pl.semaphore_signal(
                barrier_sem, inc=1,
                device_id=(my ^ mask,),
                device_id_type=pl.DeviceIdType.MESH,
            )
        pl.semaphore_wait(barrier_sem, len(_MASKS))

        xv = x_ref[:, :]
        gate = jnp.dot(xv, wg_ref[:, :], preferred_element_type=jnp.float32)
        up = jnp.dot(xv, wu_ref[:, :], preferred_element_type=jnp.float32)
        h = gate * (up * jax.nn.sigmoid(up))
        out_ref[:, :] = jnp.dot(h, wd_ref[:, :],
                                preferred_element_type=jnp.float32)

        def make_rdma(s, r):
            rows = _CHUNKS[s][1]
            partner = my ^ _SCHED[s][r]
            return pltpu.make_async_remote_copy(
                src_ref=stage_ref.at[s, r, pl.ds(0, rows), :],
                dst_ref=comm_ref.at[s, r, pl.ds(0, rows), :],
                send_sem=send_sems.at[s, r],
                recv_sem=recv_sems.at[s, r],
                device_id=(partner,),
                device_id_type=pl.DeviceIdType.MESH,
            )

        rdma = [[None] * _N_ROUNDS for _ in range(_N_STREAMS)]
        for s, (off, rows) in enumerate(_CHUNKS):
            stage_ref[s, 0, pl.ds(0, rows), :] = (
                out_ref[pl.ds(off, rows), :].astype(jnp.bfloat16))
            rdma[s][0] = make_rdma(s, 0)
            rdma[s][0].start()

        for r in range(_N_ROUNDS):
            for s, (off, rows) in enumerate(_CHUNKS):
                rdma[s][r].wait()
                acc = (out_ref[pl.ds(off, rows), :]
                       + comm_ref[s, r, pl.ds(0, rows), :].astype(jnp.float32))
                out_ref[pl.ds(off, rows), :] = acc
                if r + 1 < _N_ROUNDS:
                    stage_ref[s, r + 1, pl.ds(0, rows), :] = (
                        acc.astype(jnp.bfloat16))
                    rdma[s][r + 1] = make_rdma(s, r + 1)
                    rdma[s][r + 1].start()

    return pl.pallas_call(
        body,
        out_shape=jax.ShapeDtypeStruct((m, d), jnp.float32),
        in_specs=[pl.BlockSpec(memory_space=pltpu.VMEM)] * 4,
        out_specs=pl.BlockSpec(memory_space=pltpu.VMEM),
        scratch_shapes=[
            pltpu.VMEM((_N_STREAMS, _N_ROUNDS, _MAX_ROWS, d), jnp.bfloat16),
            pltpu.VMEM((_N_STREAMS, _N_ROUNDS, _MAX_ROWS, d), jnp.bfloat16),
            pltpu.SemaphoreType.DMA((_N_STREAMS, _N_ROUNDS)),
            pltpu.SemaphoreType.DMA((_N_STREAMS, _N_ROUNDS)),
        ],
        compiler_params=pltpu.CompilerParams(collective_id=0),
    )(x, Wg, Wu, Wd)


# device time: 5041 ns/iter; 3.1504x vs baseline; 3.1504x over previous
import jax
import jax.numpy as jnp
from jax import lax
from jax.experimental import pallas as pl
from jax.experimental.pallas import tpu as pltpu

N_DEV = 8
_MASKS = (1, 3, 4)
_CHUNKS = ((0, 96), (96, 80), (176, 80))
_SCHED = ((1, 3, 4), (3, 4, 1), (4, 1, 3))
_N_STREAMS = len(_SCHED)
_N_ROUNDS = len(_MASKS)
_MAX_ROWS = max(r for _, r in _CHUNKS)


def kernel(x, Wg, Wu, Wd):
    m, k = x.shape
    hdim = Wg.shape[1]
    d = Wd.shape[1]

    def body(x_hbm, wg_hbm, wu_hbm, wd_hbm, out_ref,
             xv_ref, wg_ref, wu_ref, wd_ref,
             stage_ref, comm_ref, load_sems, send_sems, recv_sems):
        my = lax.axis_index("i")

        loads = [
            pltpu.make_async_copy(src, dst, load_sems.at[i])
            for i, (src, dst) in enumerate([
                (x_hbm, xv_ref), (wg_hbm, wg_ref),
                (wu_hbm, wu_ref), (wd_hbm, wd_ref),
            ])
        ]
        for ld in loads:
            ld.start()

        barrier_sem = pltpu.get_barrier_semaphore()
        for mask in _MASKS:
            pl.semaphore_signal(
                barrier_sem, inc=1,
                device_id=(my ^ mask,),
                device_id_type=pl.DeviceIdType.MESH,
            )
        pl.semaphore_wait(barrier_sem, len(_MASKS))
        for ld in loads:
            ld.wait()

        xv = xv_ref[:, :]
        gate = jnp.dot(xv, wg_ref[:, :], preferred_element_type=jnp.float32)
        up = jnp.dot(xv, wu_ref[:, :], preferred_element_type=jnp.float32)
        h = gate * (up * jax.nn.sigmoid(up))
        out_ref[:, :] = jnp.dot(h, wd_ref[:, :],
                                preferred_element_type=jnp.float32)

        def make_rdma(s, r):
            rows = _CHUNKS[s][1]
            partner = my ^ _SCHED[s][r]
            return pltpu.make_async_remote_copy(
                src_ref=stage_ref.at[s, r, pl.ds(0, rows), :],
                dst_ref=comm_ref.at[s, r, pl.ds(0, rows), :],
                send_sem=send_sems.at[s, r],
                recv_sem=recv_sems.at[s, r],
                device_id=(partner,),
                device_id_type=pl.DeviceIdType.MESH,
            )

        rdma = [[None] * _N_ROUNDS for _ in range(_N_STREAMS)]
        for s, (off, rows) in enumerate(_CHUNKS):
            stage_ref[s, 0, pl.ds(0, rows), :] = (
                out_ref[pl.ds(off, rows), :].astype(jnp.bfloat16))
            rdma[s][0] = make_rdma(s, 0)
            rdma[s][0].start()

        for r in range(_N_ROUNDS):
            for s, (off, rows) in enumerate(_CHUNKS):
                rdma[s][r].wait()
                acc = (out_ref[pl.ds(off, rows), :]
                       + comm_ref[s, r, pl.ds(0, rows), :].astype(jnp.float32))
                out_ref[pl.ds(off, rows), :] = acc
                if r + 1 < _N_ROUNDS:
                    stage_ref[s, r + 1, pl.ds(0, rows), :] = (
                        acc.astype(jnp.bfloat16))
                    rdma[s][r + 1] = make_rdma(s, r + 1)
                    rdma[s][r + 1].start()

    return pl.pallas_call(
        body,
        out_shape=jax.ShapeDtypeStruct((m, d), jnp.float32),
        in_specs=[pl.BlockSpec(memory_space=pl.ANY)] * 4,
        out_specs=pl.BlockSpec(memory_space=pltpu.VMEM),
        scratch_shapes=[
            pltpu.VMEM((m, k), jnp.float32),
            pltpu.VMEM((k, hdim), jnp.float32),
            pltpu.VMEM((k, hdim), jnp.float32),
            pltpu.VMEM((hdim, d), jnp.float32),
            pltpu.VMEM((_N_STREAMS, _N_ROUNDS, _MAX_ROWS, d), jnp.bfloat16),
            pltpu.VMEM((_N_STREAMS, _N_ROUNDS, _MAX_ROWS, d), jnp.bfloat16),
            pltpu.SemaphoreType.DMA((4,)),
            pltpu.SemaphoreType.DMA((_N_STREAMS, _N_ROUNDS)),
            pltpu.SemaphoreType.DMA((_N_STREAMS, _N_ROUNDS)),
        ],
        compiler_params=pltpu.CompilerParams(collective_id=0),
    )(x, Wg, Wu, Wd)
